# baseline (speedup 1.0000x reference)
import jax
import jax.numpy as jnp
from jax import lax
from jax.experimental import pallas as pl
from jax.experimental.pallas import tpu as pltpu
from jax.experimental.pallas import tpu_sc as plsc

USER_COUNT = 25000
ITEM_COUNT = 25000
N = USER_COUNT + ITEM_COUNT
EMB = 64
NNZ = 800000
N_LAYERS = 3

NC = 2
NS = 16
LANES = 16

HALF = N // NC
HALFP = 25024
W = 128
SUBW = 16
CHUNK = W * SUBW
NCHUNK = 25
EPT = CHUNK * NCHUNK
NNZ_PAD = EPT * NS
EROWS = NNZ_PAD // W
TROWS = EPT // W

_mesh = plsc.VectorSubcoreMesh(
    core_axis_name="c", subcore_axis_name="s", num_cores=NC, num_subcores=NS
)


def _layer_body(ego, cols, rows, vals, out, acc, colc, rowc, valc, idxc, gb0,
                gb1, gsem0, gsem1, ssem0, ssem1):
  c = lax.axis_index("c")
  s = lax.axis_index("s")
  lo = c * HALF
  zero16 = jnp.zeros((LANES,), jnp.float32)

  @pl.loop(0, W)
  def _zero(r):
    for k in range(EMB // LANES):
      gb0[r, pl.ds(k * LANES, LANES)] = zero16

  tb = s * 1564
  for j in range(12):
    pltpu.sync_copy(gb0, acc.at[pl.ds(tb + j * W, W)])
  pltpu.sync_copy(gb0.at[pl.ds(0, 28)], acc.at[pl.ds(tb + 12 * W, 28)])
  plsc.subcore_barrier()

  gbufs = (gb0, gb1)
  gsems = (gsem0, gsem1)
  ssems = (ssem0, ssem1)

  @pl.loop(0, NCHUNK)
  def _chunk(ci):
    cc = ci + c * (NCHUNK // 2)
    cc = jnp.where(cc >= NCHUNK, cc - NCHUNK, cc)
    r0 = s * TROWS + cc * SUBW
    pltpu.sync_copy(cols.at[pl.ds(r0, SUBW)], colc)
    pltpu.sync_copy(rows.at[pl.ds(r0, SUBW)], rowc)
    pltpu.sync_copy(vals.at[pl.ds(r0, SUBW)], valc)

    gd = [None] * SUBW
    sd = [None] * SUBW
    gd[0] = pltpu.async_copy(ego.at[colc.at[0]], gb0, gsem0)

    @plsc.parallel_loop(0, CHUNK // LANES, unroll=2)
    def _prep(g):
      w8 = g >> 3
      sl = pl.ds((g & 7) * LANES, LANES)
      r16 = rowc[w8, sl]
      inh = (r16 >= lo) & (r16 < lo + HALF)
      idxc[w8, sl] = jnp.where(inh, r16 - lo, r16 & 0x3FFF)
      valc[w8, sl] = jnp.where(inh, valc[w8, sl], zero16)

    for w in range(SUBW):
      p = w & 1
      if w + 1 < SUBW:
        q = (w + 1) & 1
        if w + 1 >= 2:
          sd[w - 1].wait()
        gd[w + 1] = pltpu.async_copy(ego.at[colc.at[w + 1]], gbufs[q],
                                     gsems[q])
      gd[w].wait()

      gb = gbufs[p]

      @plsc.parallel_loop(0, W // LANES, unroll=4)
      def _scale(g):
        v16 = valc[w, pl.ds(g * LANES, LANES)]
        for e in range(LANES):
          v = lax.broadcast(v16[e], (LANES,))
          r = g * LANES + e
          for k in range(EMB // LANES):
            sl = pl.ds(k * LANES, LANES)
            gb[r, sl] = gb[r, sl] * v

      sd[w] = pltpu.async_copy(gb, acc.at[idxc.at[w]], ssems[p], add=True)

    sd[SUBW - 2].wait()
    sd[SUBW - 1].wait()

  plsc.subcore_barrier()

  @pl.when(s < 15)
  def _wb_lo():
    st = s * 1568
    pltpu.sync_copy(acc.at[pl.ds(st, 1568)], out.at[pl.ds(lo + st, 1568)])

  @pl.when(s == 15)
  def _wb_hi():
    st = 15 * 1568
    pltpu.sync_copy(acc.at[pl.ds(st, 1480)], out.at[pl.ds(lo + st, 1480)])


_layer = pl.kernel(
    _layer_body,
    out_type=jax.ShapeDtypeStruct((N, EMB), jnp.float32),
    mesh=_mesh,
    compiler_params=pltpu.CompilerParams(use_tc_tiling_on_sc=False),
    scratch_types=[
        pltpu.VMEM_SHARED((HALFP, EMB), jnp.float32),
        pltpu.VMEM((SUBW, W), jnp.int32),
        pltpu.VMEM((SUBW, W), jnp.int32),
        pltpu.VMEM((SUBW, W), jnp.float32),
        pltpu.VMEM((SUBW, W), jnp.int32),
        pltpu.VMEM((W, EMB), jnp.float32),
        pltpu.VMEM((W, EMB), jnp.float32),
        pltpu.SemaphoreType.DMA,
        pltpu.SemaphoreType.DMA,
        pltpu.SemaphoreType.DMA,
        pltpu.SemaphoreType.DMA,
    ],
)

P = 256


def _final_body(e0, e1, e2, e3, idx, out, ib0, ib1, b0, b1, b2, b3, sem0,
                sem1, sem2, sem3):
  c = lax.axis_index("c")
  s = lax.axis_index("s")
  wid = s * NC + c
  base = wid * P

  pltpu.sync_copy(idx.at[pl.ds(base, P // 2)], ib0)
  pltpu.sync_copy(idx.at[pl.ds(base + P // 2, P // 2)], ib1)

  descs = []
  for tab, buf, sem in ((e0, b0, sem0), (e1, b1, sem1), (e2, b2, sem2),
                        (e3, b3, sem3)):
    descs.append(pltpu.async_copy(tab.at[ib0], buf.at[pl.ds(0, P // 2)], sem))
    descs.append(
        pltpu.async_copy(tab.at[ib1], buf.at[pl.ds(P // 2, P // 2)], sem))
  for d in descs:
    d.wait()

  quarter = jnp.full((LANES,), 0.25, jnp.float32)

  @pl.loop(0, P)
  def _avg(r):
    for k in range(EMB // LANES):
      sl = pl.ds(k * LANES, LANES)
      acc16 = (b0[r, sl] + b1[r, sl]) + (b2[r, sl] + b3[r, sl])
      b0[r, sl] = acc16 * quarter

  pltpu.sync_copy(b0, out.at[pl.ds(base, P)])


_final = pl.kernel(
    _final_body,
    out_type=jax.ShapeDtypeStruct((2 * 4096, EMB), jnp.float32),
    mesh=_mesh,
    compiler_params=pltpu.CompilerParams(use_tc_tiling_on_sc=False),
    scratch_types=[
        pltpu.VMEM((P // 2,), jnp.int32),
        pltpu.VMEM((P // 2,), jnp.int32),
        pltpu.VMEM((P, EMB), jnp.float32),
        pltpu.VMEM((P, EMB), jnp.float32),
        pltpu.VMEM((P, EMB), jnp.float32),
        pltpu.VMEM((P, EMB), jnp.float32),
        pltpu.SemaphoreType.DMA,
        pltpu.SemaphoreType.DMA,
        pltpu.SemaphoreType.DMA,
        pltpu.SemaphoreType.DMA,
    ],
)


@jax.jit
def kernel(users, items, user_emb, item_emb, adj_row, adj_col, adj_val):
  ego0 = jnp.concatenate([user_emb, item_emb], axis=0)

  padn = NNZ_PAD - NNZ
  cols = jnp.concatenate(
      [adj_col.astype(jnp.int32),
       jnp.zeros((padn,), jnp.int32)]).reshape(EROWS, W)
  rows = jnp.concatenate([
      adj_row.astype(jnp.int32),
      (jnp.arange(padn, dtype=jnp.int32) * 61) % N,
  ]).reshape(EROWS, W)
  vals = jnp.concatenate([adj_val,
                          jnp.zeros((padn,), jnp.float32)]).reshape(EROWS, W)

  e1 = _layer(ego0, cols, rows, vals)
  e2 = _layer(e1, cols, rows, vals)
  e3 = _layer(e2, cols, rows, vals)

  nb = users.shape[0]
  idx_all = jnp.concatenate(
      [users.astype(jnp.int32),
       items.astype(jnp.int32) + USER_COUNT])
  out = _final(ego0, e1, e2, e3, idx_all)
  return out[:nb], out[nb:]


# speedup vs baseline: 1.8388x; 1.8388x over previous
import jax
import jax.numpy as jnp
from jax import lax
from jax.experimental import pallas as pl
from jax.experimental.pallas import tpu as pltpu
from jax.experimental.pallas import tpu_sc as plsc

USER_COUNT = 25000
ITEM_COUNT = 25000
N = USER_COUNT + ITEM_COUNT
EMB = 64
NNZ = 800000
N_LAYERS = 3

NC = 2
NS = 16
LANES = 16

HALF = N // NC
HALFP = 25024
W = 128
SUBW = 8
CHUNK = W * SUBW
NCHUNK = 49
EPT = CHUNK * NCHUNK
NNZ_PAD = EPT * NS
EROWS = NNZ_PAD // W
TROWS = EPT // W

_mesh = plsc.VectorSubcoreMesh(
    core_axis_name="c", subcore_axis_name="s", num_cores=NC, num_subcores=NS
)


def _layer_body(ego, cols, rows, vals, out, acc, colc, rowc, valc, idxc, gb0,
                gb1, gsem0, gsem1, ssem0, ssem1):
  c = lax.axis_index("c")
  s = lax.axis_index("s")
  lo = c * HALF
  zero16 = jnp.zeros((LANES,), jnp.float32)

  @pl.loop(0, W)
  def _zero(r):
    for k in range(EMB // LANES):
      gb0[r, pl.ds(k * LANES, LANES)] = zero16

  tb = s * 1564
  for j in range(12):
    pltpu.sync_copy(gb0, acc.at[pl.ds(tb + j * W, W)])
  pltpu.sync_copy(gb0.at[pl.ds(0, 28)], acc.at[pl.ds(tb + 12 * W, 28)])
  plsc.subcore_barrier()

  gbufs = (gb0, gb1)
  gsems = (gsem0, gsem1)
  ssems = (ssem0, ssem1)

  @pl.loop(0, NCHUNK)
  def _chunk(ci):
    cc = ci + c * (NCHUNK // 2)
    cc = jnp.where(cc >= NCHUNK, cc - NCHUNK, cc)
    r0 = s * TROWS + cc * SUBW
    pltpu.sync_copy(cols.at[pl.ds(r0, SUBW)], colc)
    pltpu.sync_copy(rows.at[pl.ds(r0, SUBW)], rowc)
    pltpu.sync_copy(vals.at[pl.ds(r0, SUBW)], valc)

    gd = [None] * SUBW
    sd = [None] * SUBW
    gd[0] = pltpu.async_copy(ego.at[colc.at[0]], gb0, gsem0)

    @plsc.parallel_loop(0, CHUNK // LANES, unroll=2)
    def _prep(g):
      w8 = g >> 3
      sl = pl.ds((g & 7) * LANES, LANES)
      r16 = rowc[w8, sl]
      inh = (r16 >= lo) & (r16 < lo + HALF)
      idxc[w8, sl] = jnp.where(inh, r16 - lo, r16 & 0x3FFF)
      valc[w8, sl] = jnp.where(inh, valc[w8, sl], zero16)

    for w in range(SUBW):
      p = w & 1
      if w + 1 < SUBW:
        q = (w + 1) & 1
        if w + 1 >= 2:
          sd[w - 1].wait()
        gd[w + 1] = pltpu.async_copy(ego.at[colc.at[w + 1]], gbufs[q],
                                     gsems[q])
      gd[w].wait()

      gb = gbufs[p]

      @plsc.parallel_loop(0, W // LANES, unroll=4)
      def _scale(g):
        v16 = valc[w, pl.ds(g * LANES, LANES)]
        for e in range(LANES):
          v = lax.broadcast(v16[e], (LANES,))
          r = g * LANES + e
          for k in range(EMB // LANES):
            sl = pl.ds(k * LANES, LANES)
            gb[r, sl] = gb[r, sl] * v

      sd[w] = pltpu.async_copy(gb, acc.at[idxc.at[w]], ssems[p], add=True)

    sd[SUBW - 2].wait()
    sd[SUBW - 1].wait()

  plsc.subcore_barrier()

  @pl.when(s < 15)
  def _wb_lo():
    st = s * 1568
    pltpu.sync_copy(acc.at[pl.ds(st, 1568)], out.at[pl.ds(lo + st, 1568)])

  @pl.when(s == 15)
  def _wb_hi():
    st = 15 * 1568
    pltpu.sync_copy(acc.at[pl.ds(st, 1480)], out.at[pl.ds(lo + st, 1480)])


_layer = pl.kernel(
    _layer_body,
    out_type=jax.ShapeDtypeStruct((N, EMB), jnp.float32),
    mesh=_mesh,
    compiler_params=pltpu.CompilerParams(use_tc_tiling_on_sc=False),
    scratch_types=[
        pltpu.VMEM_SHARED((HALFP, EMB), jnp.float32),
        pltpu.VMEM((SUBW, W), jnp.int32),
        pltpu.VMEM((SUBW, W), jnp.int32),
        pltpu.VMEM((SUBW, W), jnp.float32),
        pltpu.VMEM((SUBW, W), jnp.int32),
        pltpu.VMEM((W, EMB), jnp.float32),
        pltpu.VMEM((W, EMB), jnp.float32),
        pltpu.SemaphoreType.DMA,
        pltpu.SemaphoreType.DMA,
        pltpu.SemaphoreType.DMA,
        pltpu.SemaphoreType.DMA,
    ],
)

P = 256


def _final_body(e0, e1, e2, e3, idx, out, ib0, ib1, b0, b1, b2, b3, sem0,
                sem1, sem2, sem3):
  c = lax.axis_index("c")
  s = lax.axis_index("s")
  wid = s * NC + c
  base = wid * P

  pltpu.sync_copy(idx.at[pl.ds(base, P // 2)], ib0)
  pltpu.sync_copy(idx.at[pl.ds(base + P // 2, P // 2)], ib1)

  descs = []
  for tab, buf, sem in ((e0, b0, sem0), (e1, b1, sem1), (e2, b2, sem2),
                        (e3, b3, sem3)):
    descs.append(pltpu.async_copy(tab.at[ib0], buf.at[pl.ds(0, P // 2)], sem))
    descs.append(
        pltpu.async_copy(tab.at[ib1], buf.at[pl.ds(P // 2, P // 2)], sem))
  for d in descs:
    d.wait()

  quarter = jnp.full((LANES,), 0.25, jnp.float32)

  @pl.loop(0, P)
  def _avg(r):
    for k in range(EMB // LANES):
      sl = pl.ds(k * LANES, LANES)
      acc16 = (b0[r, sl] + b1[r, sl]) + (b2[r, sl] + b3[r, sl])
      b0[r, sl] = acc16 * quarter

  pltpu.sync_copy(b0, out.at[pl.ds(base, P)])


_final = pl.kernel(
    _final_body,
    out_type=jax.ShapeDtypeStruct((2 * 4096, EMB), jnp.float32),
    mesh=_mesh,
    compiler_params=pltpu.CompilerParams(use_tc_tiling_on_sc=False),
    scratch_types=[
        pltpu.VMEM((P // 2,), jnp.int32),
        pltpu.VMEM((P // 2,), jnp.int32),
        pltpu.VMEM((P, EMB), jnp.float32),
        pltpu.VMEM((P, EMB), jnp.float32),
        pltpu.VMEM((P, EMB), jnp.float32),
        pltpu.VMEM((P, EMB), jnp.float32),
        pltpu.SemaphoreType.DMA,
        pltpu.SemaphoreType.DMA,
        pltpu.SemaphoreType.DMA,
        pltpu.SemaphoreType.DMA,
    ],
)


@jax.jit
def kernel(users, items, user_emb, item_emb, adj_row, adj_col, adj_val):
  ego0 = jnp.concatenate([user_emb, item_emb], axis=0)

  padn = NNZ_PAD - NNZ
  cols = jnp.concatenate(
      [adj_col.astype(jnp.int32),
       jnp.zeros((padn,), jnp.int32)]).reshape(EROWS, W)
  rows = jnp.concatenate([
      adj_row.astype(jnp.int32),
      (jnp.arange(padn, dtype=jnp.int32) * 61) % N,
  ]).reshape(EROWS, W)
  vals = jnp.concatenate([adj_val,
                          jnp.zeros((padn,), jnp.float32)]).reshape(EROWS, W)

  e1 = _layer(ego0, cols, rows, vals)
  e2 = _layer(e1, cols, rows, vals)
  e3 = _layer(e2, cols, rows, vals)

  nb = users.shape[0]
  idx_all = jnp.concatenate(
      [users.astype(jnp.int32),
       items.astype(jnp.int32) + USER_COUNT])
  out = _final(ego0, e1, e2, e3, idx_all)
  return out[:nb], out[nb:]


# speedup vs baseline: 1.8554x; 1.0090x over previous
import jax
import jax.numpy as jnp
from jax import lax
from jax.experimental import pallas as pl
from jax.experimental.pallas import tpu as pltpu
from jax.experimental.pallas import tpu_sc as plsc

USER_COUNT = 25000
ITEM_COUNT = 25000
N = USER_COUNT + ITEM_COUNT
EMB = 64
NNZ = 800000
N_LAYERS = 3

NC = 2
NS = 16
LANES = 16

HALF = N // NC
HALFP = 25024
W = 128
SUBW = 8
CHUNK = W * SUBW
NCHUNK = 49
EPT = CHUNK * NCHUNK
NNZ_PAD = EPT * NS
EROWS = NNZ_PAD // W
TROWS = EPT // W

_mesh = plsc.VectorSubcoreMesh(
    core_axis_name="c", subcore_axis_name="s", num_cores=NC, num_subcores=NS
)


def _layer_body(ego, cols, rows, vals, out, acc, colc, rowc, valc, idxc, gb0,
                gb1, gsem0, gsem1, ssem0, ssem1):
  c = lax.axis_index("c")
  s = lax.axis_index("s")
  lo = c * HALF
  zero16 = jnp.zeros((LANES,), jnp.float32)

  @pl.loop(0, W)
  def _zero(r):
    for k in range(EMB // LANES):
      gb0[r, pl.ds(k * LANES, LANES)] = zero16

  tb = s * 1564
  for j in range(12):
    pltpu.sync_copy(gb0, acc.at[pl.ds(tb + j * W, W)])
  pltpu.sync_copy(gb0.at[pl.ds(0, 28)], acc.at[pl.ds(tb + 12 * W, 28)])
  plsc.subcore_barrier()

  gbufs = (gb0, gb1)
  gsems = (gsem0, gsem1)
  ssems = (ssem0, ssem1)

  @pl.loop(0, NCHUNK)
  def _chunk(ci):
    cc = ci + c * (NCHUNK // 2)
    cc = jnp.where(cc >= NCHUNK, cc - NCHUNK, cc)
    r0 = s * TROWS + cc * SUBW
    pltpu.sync_copy(cols.at[pl.ds(r0, SUBW)], colc)
    pltpu.sync_copy(rows.at[pl.ds(r0, SUBW)], rowc)
    pltpu.sync_copy(vals.at[pl.ds(r0, SUBW)], valc)

    gd = [None] * SUBW
    sd = [None] * SUBW
    gd[0] = pltpu.async_copy(ego.at[colc.at[0]], gb0, gsem0)

    @plsc.parallel_loop(0, CHUNK // LANES, unroll=2)
    def _prep(g):
      w8 = g >> 3
      sl = pl.ds((g & 7) * LANES, LANES)
      r16 = rowc[w8, sl]
      inh = (r16 >= lo) & (r16 < lo + HALF)
      idxc[w8, sl] = jnp.where(inh, r16 - lo, r16 & 0x3FFF)
      valc[w8, sl] = jnp.where(inh, valc[w8, sl], zero16)

    for w in range(SUBW):
      p = w & 1
      if w + 1 < SUBW:
        q = (w + 1) & 1
        if w + 1 >= 2:
          sd[w - 1].wait()
        gd[w + 1] = pltpu.async_copy(ego.at[colc.at[w + 1]], gbufs[q],
                                     gsems[q])
      gd[w].wait()

      gb = gbufs[p]

      @plsc.parallel_loop(0, W // LANES, unroll=2)
      def _scale(g):
        v16 = valc[w, pl.ds(g * LANES, LANES)]
        for e in range(LANES):
          v = lax.broadcast(v16[e], (LANES,))
          r = g * LANES + e
          for k in range(EMB // LANES):
            sl = pl.ds(k * LANES, LANES)
            gb[r, sl] = gb[r, sl] * v

      sd[w] = pltpu.async_copy(gb, acc.at[idxc.at[w]], ssems[p], add=True)

    sd[SUBW - 2].wait()
    sd[SUBW - 1].wait()

  plsc.subcore_barrier()

  @pl.when(s < 15)
  def _wb_lo():
    st = s * 1568
    pltpu.sync_copy(acc.at[pl.ds(st, 1568)], out.at[pl.ds(lo + st, 1568)])

  @pl.when(s == 15)
  def _wb_hi():
    st = 15 * 1568
    pltpu.sync_copy(acc.at[pl.ds(st, 1480)], out.at[pl.ds(lo + st, 1480)])


_layer = pl.kernel(
    _layer_body,
    out_type=jax.ShapeDtypeStruct((N, EMB), jnp.float32),
    mesh=_mesh,
    compiler_params=pltpu.CompilerParams(use_tc_tiling_on_sc=False),
    scratch_types=[
        pltpu.VMEM_SHARED((HALFP, EMB), jnp.float32),
        pltpu.VMEM((SUBW, W), jnp.int32),
        pltpu.VMEM((SUBW, W), jnp.int32),
        pltpu.VMEM((SUBW, W), jnp.float32),
        pltpu.VMEM((SUBW, W), jnp.int32),
        pltpu.VMEM((W, EMB), jnp.float32),
        pltpu.VMEM((W, EMB), jnp.float32),
        pltpu.SemaphoreType.DMA,
        pltpu.SemaphoreType.DMA,
        pltpu.SemaphoreType.DMA,
        pltpu.SemaphoreType.DMA,
    ],
)

P = 256


def _final_body(e0, e1, e2, e3, idx, out, ib0, ib1, b0, b1, b2, b3, sem0,
                sem1, sem2, sem3):
  c = lax.axis_index("c")
  s = lax.axis_index("s")
  wid = s * NC + c
  base = wid * P

  pltpu.sync_copy(idx.at[pl.ds(base, P // 2)], ib0)
  pltpu.sync_copy(idx.at[pl.ds(base + P // 2, P // 2)], ib1)

  descs = []
  for tab, buf, sem in ((e0, b0, sem0), (e1, b1, sem1), (e2, b2, sem2),
                        (e3, b3, sem3)):
    descs.append(pltpu.async_copy(tab.at[ib0], buf.at[pl.ds(0, P // 2)], sem))
    descs.append(
        pltpu.async_copy(tab.at[ib1], buf.at[pl.ds(P // 2, P // 2)], sem))
  for d in descs:
    d.wait()

  quarter = jnp.full((LANES,), 0.25, jnp.float32)

  @pl.loop(0, P)
  def _avg(r):
    for k in range(EMB // LANES):
      sl = pl.ds(k * LANES, LANES)
      acc16 = (b0[r, sl] + b1[r, sl]) + (b2[r, sl] + b3[r, sl])
      b0[r, sl] = acc16 * quarter

  pltpu.sync_copy(b0, out.at[pl.ds(base, P)])


_final = pl.kernel(
    _final_body,
    out_type=jax.ShapeDtypeStruct((2 * 4096, EMB), jnp.float32),
    mesh=_mesh,
    compiler_params=pltpu.CompilerParams(use_tc_tiling_on_sc=False),
    scratch_types=[
        pltpu.VMEM((P // 2,), jnp.int32),
        pltpu.VMEM((P // 2,), jnp.int32),
        pltpu.VMEM((P, EMB), jnp.float32),
        pltpu.VMEM((P, EMB), jnp.float32),
        pltpu.VMEM((P, EMB), jnp.float32),
        pltpu.VMEM((P, EMB), jnp.float32),
        pltpu.SemaphoreType.DMA,
        pltpu.SemaphoreType.DMA,
        pltpu.SemaphoreType.DMA,
        pltpu.SemaphoreType.DMA,
    ],
)


@jax.jit
def kernel(users, items, user_emb, item_emb, adj_row, adj_col, adj_val):
  ego0 = jnp.concatenate([user_emb, item_emb], axis=0)

  padn = NNZ_PAD - NNZ
  cols = jnp.concatenate(
      [adj_col.astype(jnp.int32),
       jnp.zeros((padn,), jnp.int32)]).reshape(EROWS, W)
  rows = jnp.concatenate([
      adj_row.astype(jnp.int32),
      (jnp.arange(padn, dtype=jnp.int32) * 61) % N,
  ]).reshape(EROWS, W)
  vals = jnp.concatenate([adj_val,
                          jnp.zeros((padn,), jnp.float32)]).reshape(EROWS, W)

  e1 = _layer(ego0, cols, rows, vals)
  e2 = _layer(e1, cols, rows, vals)
  e3 = _layer(e2, cols, rows, vals)

  nb = users.shape[0]
  idx_all = jnp.concatenate(
      [users.astype(jnp.int32),
       items.astype(jnp.int32) + USER_COUNT])
  out = _final(ego0, e1, e2, e3, idx_all)
  return out[:nb], out[nb:]


# speedup vs baseline: 1.8653x; 1.0054x over previous
import jax
import jax.numpy as jnp
from jax import lax
from jax.experimental import pallas as pl
from jax.experimental.pallas import tpu as pltpu
from jax.experimental.pallas import tpu_sc as plsc

USER_COUNT = 25000
ITEM_COUNT = 25000
N = USER_COUNT + ITEM_COUNT
EMB = 64
NNZ = 800000
N_LAYERS = 3

NC = 2
NS = 16
LANES = 16

HALF = N // NC
HALFP = 25024
W = 128
SUBW = 8
CHUNK = W * SUBW
NCHUNK = 49
EPT = CHUNK * NCHUNK
NNZ_PAD = EPT * NS
EROWS = NNZ_PAD // W
TROWS = EPT // W

_mesh = plsc.VectorSubcoreMesh(
    core_axis_name="c", subcore_axis_name="s", num_cores=NC, num_subcores=NS
)


def _layer_body(ego, cols, rows, vals, out, acc, colc, rowc, valc, idxc, gb0,
                gb1, gsem0, gsem1, ssem0, ssem1):
  c = lax.axis_index("c")
  s = lax.axis_index("s")
  lo = c * HALF
  zero16 = jnp.zeros((LANES,), jnp.float32)

  @pl.loop(0, W)
  def _zero(r):
    for k in range(EMB // LANES):
      gb0[r, pl.ds(k * LANES, LANES)] = zero16

  tb = s * 1564
  for j in range(12):
    pltpu.sync_copy(gb0, acc.at[pl.ds(tb + j * W, W)])
  pltpu.sync_copy(gb0.at[pl.ds(0, 28)], acc.at[pl.ds(tb + 12 * W, 28)])
  plsc.subcore_barrier()

  gbufs = (gb0, gb1)
  gsems = (gsem0, gsem1)
  ssems = (ssem0, ssem1)

  @pl.loop(0, NCHUNK)
  def _chunk(ci):
    cc = ci + c * (NCHUNK // 2)
    cc = jnp.where(cc >= NCHUNK, cc - NCHUNK, cc)
    r0 = s * TROWS + cc * SUBW
    pltpu.sync_copy(cols.at[pl.ds(r0, SUBW)], colc)
    pltpu.sync_copy(rows.at[pl.ds(r0, SUBW)], rowc)
    pltpu.sync_copy(vals.at[pl.ds(r0, SUBW)], valc)

    gd = [None] * SUBW
    sd = [None] * SUBW
    gd[0] = pltpu.async_copy(ego.at[colc.at[0]], gb0, gsem0)

    @plsc.parallel_loop(0, CHUNK // LANES)
    def _prep(g):
      w8 = g >> 3
      sl = pl.ds((g & 7) * LANES, LANES)
      r16 = rowc[w8, sl]
      inh = (r16 >= lo) & (r16 < lo + HALF)
      idxc[w8, sl] = jnp.where(inh, r16 - lo, r16 & 0x3FFF)
      valc[w8, sl] = jnp.where(inh, valc[w8, sl], zero16)

    for w in range(SUBW):
      p = w & 1
      if w + 1 < SUBW:
        q = (w + 1) & 1
        if w + 1 >= 2:
          sd[w - 1].wait()
        gd[w + 1] = pltpu.async_copy(ego.at[colc.at[w + 1]], gbufs[q],
                                     gsems[q])
      gd[w].wait()

      gb = gbufs[p]

      @plsc.parallel_loop(0, W // LANES, unroll=2)
      def _scale(g):
        v16 = valc[w, pl.ds(g * LANES, LANES)]
        for e in range(LANES):
          v = lax.broadcast(v16[e], (LANES,))
          r = g * LANES + e
          for k in range(EMB // LANES):
            sl = pl.ds(k * LANES, LANES)
            gb[r, sl] = gb[r, sl] * v

      sd[w] = pltpu.async_copy(gb, acc.at[idxc.at[w]], ssems[p], add=True)

    sd[SUBW - 2].wait()
    sd[SUBW - 1].wait()

  plsc.subcore_barrier()

  @pl.when(s < 15)
  def _wb_lo():
    st = s * 1568
    pltpu.sync_copy(acc.at[pl.ds(st, 1568)], out.at[pl.ds(lo + st, 1568)])

  @pl.when(s == 15)
  def _wb_hi():
    st = 15 * 1568
    pltpu.sync_copy(acc.at[pl.ds(st, 1480)], out.at[pl.ds(lo + st, 1480)])


_layer = pl.kernel(
    _layer_body,
    out_type=jax.ShapeDtypeStruct((N, EMB), jnp.float32),
    mesh=_mesh,
    compiler_params=pltpu.CompilerParams(use_tc_tiling_on_sc=False),
    scratch_types=[
        pltpu.VMEM_SHARED((HALFP, EMB), jnp.float32),
        pltpu.VMEM((SUBW, W), jnp.int32),
        pltpu.VMEM((SUBW, W), jnp.int32),
        pltpu.VMEM((SUBW, W), jnp.float32),
        pltpu.VMEM((SUBW, W), jnp.int32),
        pltpu.VMEM((W, EMB), jnp.float32),
        pltpu.VMEM((W, EMB), jnp.float32),
        pltpu.SemaphoreType.DMA,
        pltpu.SemaphoreType.DMA,
        pltpu.SemaphoreType.DMA,
        pltpu.SemaphoreType.DMA,
    ],
)

P = 256


def _final_body(e0, e1, e2, e3, idx, out, ib0, ib1, b0, b1, b2, b3, sem0,
                sem1, sem2, sem3):
  c = lax.axis_index("c")
  s = lax.axis_index("s")
  wid = s * NC + c
  base = wid * P

  pltpu.sync_copy(idx.at[pl.ds(base, P // 2)], ib0)
  pltpu.sync_copy(idx.at[pl.ds(base + P // 2, P // 2)], ib1)

  descs = []
  for tab, buf, sem in ((e0, b0, sem0), (e1, b1, sem1), (e2, b2, sem2),
                        (e3, b3, sem3)):
    descs.append(pltpu.async_copy(tab.at[ib0], buf.at[pl.ds(0, P // 2)], sem))
    descs.append(
        pltpu.async_copy(tab.at[ib1], buf.at[pl.ds(P // 2, P // 2)], sem))
  for d in descs:
    d.wait()

  quarter = jnp.full((LANES,), 0.25, jnp.float32)

  @pl.loop(0, P)
  def _avg(r):
    for k in range(EMB // LANES):
      sl = pl.ds(k * LANES, LANES)
      acc16 = (b0[r, sl] + b1[r, sl]) + (b2[r, sl] + b3[r, sl])
      b0[r, sl] = acc16 * quarter

  pltpu.sync_copy(b0, out.at[pl.ds(base, P)])


_final = pl.kernel(
    _final_body,
    out_type=jax.ShapeDtypeStruct((2 * 4096, EMB), jnp.float32),
    mesh=_mesh,
    compiler_params=pltpu.CompilerParams(use_tc_tiling_on_sc=False),
    scratch_types=[
        pltpu.VMEM((P // 2,), jnp.int32),
        pltpu.VMEM((P // 2,), jnp.int32),
        pltpu.VMEM((P, EMB), jnp.float32),
        pltpu.VMEM((P, EMB), jnp.float32),
        pltpu.VMEM((P, EMB), jnp.float32),
        pltpu.VMEM((P, EMB), jnp.float32),
        pltpu.SemaphoreType.DMA,
        pltpu.SemaphoreType.DMA,
        pltpu.SemaphoreType.DMA,
        pltpu.SemaphoreType.DMA,
    ],
)


@jax.jit
def kernel(users, items, user_emb, item_emb, adj_row, adj_col, adj_val):
  ego0 = jnp.concatenate([user_emb, item_emb], axis=0)

  padn = NNZ_PAD - NNZ
  cols = jnp.concatenate(
      [adj_col.astype(jnp.int32),
       jnp.zeros((padn,), jnp.int32)]).reshape(EROWS, W)
  rows = jnp.concatenate([
      adj_row.astype(jnp.int32),
      (jnp.arange(padn, dtype=jnp.int32) * 61) % N,
  ]).reshape(EROWS, W)
  vals = jnp.concatenate([adj_val,
                          jnp.zeros((padn,), jnp.float32)]).reshape(EROWS, W)

  e1 = _layer(ego0, cols, rows, vals)
  e2 = _layer(e1, cols, rows, vals)
  e3 = _layer(e2, cols, rows, vals)

  nb = users.shape[0]
  idx_all = jnp.concatenate(
      [users.astype(jnp.int32),
       items.astype(jnp.int32) + USER_COUNT])
  out = _final(ego0, e1, e2, e3, idx_all)
  return out[:nb], out[nb:]


# speedup vs baseline: 1.8736x; 1.0044x over previous
import jax
import jax.numpy as jnp
from jax import lax
from jax.experimental import pallas as pl
from jax.experimental.pallas import tpu as pltpu
from jax.experimental.pallas import tpu_sc as plsc

USER_COUNT = 25000
ITEM_COUNT = 25000
N = USER_COUNT + ITEM_COUNT
EMB = 64
NNZ = 800000
N_LAYERS = 3

NC = 2
NS = 16
LANES = 16

HALF = N // NC
HALFP = 25024
W = 128
SUBW = 8
CHUNK = W * SUBW
NCHUNK = 49
EPT = CHUNK * NCHUNK
NNZ_PAD = EPT * NS
EROWS = NNZ_PAD // W
TROWS = EPT // W

_mesh = plsc.VectorSubcoreMesh(
    core_axis_name="c", subcore_axis_name="s", num_cores=NC, num_subcores=NS
)


def _layer_body(ego, cols, rows, vals, out, acc, colc, rowc, valc, idxc, gb0,
                gb1, gsem0, gsem1, ssem0, ssem1):
  c = lax.axis_index("c")
  s = lax.axis_index("s")
  lo = c * HALF
  zero16 = jnp.zeros((LANES,), jnp.float32)

  @pl.loop(0, W)
  def _zero(r):
    for k in range(EMB // LANES):
      gb0[r, pl.ds(k * LANES, LANES)] = zero16

  tb = s * 1564
  for j in range(12):
    pltpu.sync_copy(gb0, acc.at[pl.ds(tb + j * W, W)])
  pltpu.sync_copy(gb0.at[pl.ds(0, 28)], acc.at[pl.ds(tb + 12 * W, 28)])
  plsc.subcore_barrier()

  gbufs = (gb0, gb1)
  gsems = (gsem0, gsem1)
  ssems = (ssem0, ssem1)

  @pl.loop(0, NCHUNK)
  def _chunk(ci):
    cc = ci + c * (NCHUNK // 2)
    cc = jnp.where(cc >= NCHUNK, cc - NCHUNK, cc)
    r0 = s * TROWS + cc * SUBW
    pltpu.sync_copy(cols.at[pl.ds(r0, SUBW)], colc)
    pltpu.sync_copy(rows.at[pl.ds(r0, SUBW)], rowc)
    pltpu.sync_copy(vals.at[pl.ds(r0, SUBW)], valc)

    gd = [None] * SUBW
    sd = [None] * SUBW
    gd[0] = pltpu.async_copy(ego.at[colc.at[0]], gb0, gsem0)

    @plsc.parallel_loop(0, CHUNK // LANES)
    def _prep(g):
      w8 = g >> 3
      sl = pl.ds((g & 7) * LANES, LANES)
      r16 = rowc[w8, sl]
      inh = (r16 >= lo) & (r16 < lo + HALF)
      idxc[w8, sl] = jnp.where(inh, r16 - lo, r16 & 0x3FFF)
      valc[w8, sl] = jnp.where(inh, valc[w8, sl], zero16)

    for w in range(SUBW):
      p = w & 1
      if w + 1 < SUBW:
        q = (w + 1) & 1
        if w + 1 >= 2:
          sd[w - 1].wait()
        gd[w + 1] = pltpu.async_copy(ego.at[colc.at[w + 1]], gbufs[q],
                                     gsems[q])
      gd[w].wait()

      gb = gbufs[p]

      @plsc.parallel_loop(0, W // LANES)
      def _scale(g):
        v16 = valc[w, pl.ds(g * LANES, LANES)]
        for e in range(LANES):
          v = lax.broadcast(v16[e], (LANES,))
          r = g * LANES + e
          for k in range(EMB // LANES):
            sl = pl.ds(k * LANES, LANES)
            gb[r, sl] = gb[r, sl] * v

      sd[w] = pltpu.async_copy(gb, acc.at[idxc.at[w]], ssems[p], add=True)

    sd[SUBW - 2].wait()
    sd[SUBW - 1].wait()

  plsc.subcore_barrier()

  @pl.when(s < 15)
  def _wb_lo():
    st = s * 1568
    pltpu.sync_copy(acc.at[pl.ds(st, 1568)], out.at[pl.ds(lo + st, 1568)])

  @pl.when(s == 15)
  def _wb_hi():
    st = 15 * 1568
    pltpu.sync_copy(acc.at[pl.ds(st, 1480)], out.at[pl.ds(lo + st, 1480)])


_layer = pl.kernel(
    _layer_body,
    out_type=jax.ShapeDtypeStruct((N, EMB), jnp.float32),
    mesh=_mesh,
    compiler_params=pltpu.CompilerParams(use_tc_tiling_on_sc=False),
    scratch_types=[
        pltpu.VMEM_SHARED((HALFP, EMB), jnp.float32),
        pltpu.VMEM((SUBW, W), jnp.int32),
        pltpu.VMEM((SUBW, W), jnp.int32),
        pltpu.VMEM((SUBW, W), jnp.float32),
        pltpu.VMEM((SUBW, W), jnp.int32),
        pltpu.VMEM((W, EMB), jnp.float32),
        pltpu.VMEM((W, EMB), jnp.float32),
        pltpu.SemaphoreType.DMA,
        pltpu.SemaphoreType.DMA,
        pltpu.SemaphoreType.DMA,
        pltpu.SemaphoreType.DMA,
    ],
)

P = 256


def _final_body(e0, e1, e2, e3, idx, out, ib0, ib1, b0, b1, b2, b3, sem0,
                sem1, sem2, sem3):
  c = lax.axis_index("c")
  s = lax.axis_index("s")
  wid = s * NC + c
  base = wid * P

  pltpu.sync_copy(idx.at[pl.ds(base, P // 2)], ib0)
  pltpu.sync_copy(idx.at[pl.ds(base + P // 2, P // 2)], ib1)

  descs = []
  for tab, buf, sem in ((e0, b0, sem0), (e1, b1, sem1), (e2, b2, sem2),
                        (e3, b3, sem3)):
    descs.append(pltpu.async_copy(tab.at[ib0], buf.at[pl.ds(0, P // 2)], sem))
    descs.append(
        pltpu.async_copy(tab.at[ib1], buf.at[pl.ds(P // 2, P // 2)], sem))
  for d in descs:
    d.wait()

  quarter = jnp.full((LANES,), 0.25, jnp.float32)

  @pl.loop(0, P)
  def _avg(r):
    for k in range(EMB // LANES):
      sl = pl.ds(k * LANES, LANES)
      acc16 = (b0[r, sl] + b1[r, sl]) + (b2[r, sl] + b3[r, sl])
      b0[r, sl] = acc16 * quarter

  pltpu.sync_copy(b0, out.at[pl.ds(base, P)])


_final = pl.kernel(
    _final_body,
    out_type=jax.ShapeDtypeStruct((2 * 4096, EMB), jnp.float32),
    mesh=_mesh,
    compiler_params=pltpu.CompilerParams(use_tc_tiling_on_sc=False),
    scratch_types=[
        pltpu.VMEM((P // 2,), jnp.int32),
        pltpu.VMEM((P // 2,), jnp.int32),
        pltpu.VMEM((P, EMB), jnp.float32),
        pltpu.VMEM((P, EMB), jnp.float32),
        pltpu.VMEM((P, EMB), jnp.float32),
        pltpu.VMEM((P, EMB), jnp.float32),
        pltpu.SemaphoreType.DMA,
        pltpu.SemaphoreType.DMA,
        pltpu.SemaphoreType.DMA,
        pltpu.SemaphoreType.DMA,
    ],
)


@jax.jit
def kernel(users, items, user_emb, item_emb, adj_row, adj_col, adj_val):
  ego0 = jnp.concatenate([user_emb, item_emb], axis=0)

  padn = NNZ_PAD - NNZ
  cols = jnp.concatenate(
      [adj_col.astype(jnp.int32),
       jnp.zeros((padn,), jnp.int32)]).reshape(EROWS, W)
  rows = jnp.concatenate([
      adj_row.astype(jnp.int32),
      (jnp.arange(padn, dtype=jnp.int32) * 61) % N,
  ]).reshape(EROWS, W)
  vals = jnp.concatenate([adj_val,
                          jnp.zeros((padn,), jnp.float32)]).reshape(EROWS, W)

  e1 = _layer(ego0, cols, rows, vals)
  e2 = _layer(e1, cols, rows, vals)
  e3 = _layer(e2, cols, rows, vals)

  nb = users.shape[0]
  idx_all = jnp.concatenate(
      [users.astype(jnp.int32),
       items.astype(jnp.int32) + USER_COUNT])
  out = _final(ego0, e1, e2, e3, idx_all)
  return out[:nb], out[nb:]
